# baseline (speedup 1.0000x reference)
import jax
import jax.numpy as jnp
from jax.experimental import pallas as pl
from jax.experimental.pallas import tpu as pltpu

N_GRAPH = 256
BLOCK = 4000


def _body(seg_ref, hT_ref, h0_ref, W1_ref, b1_ref, W2_ref, b2_ref, out_ref):
    x = jnp.concatenate([hT_ref[...], h0_ref[...]], axis=1)

    def dot(a, b):
        return jax.lax.dot_general(a, b, (((1,), (0,)), ((), ())),
                                   preferred_element_type=jnp.float32)

    p = jnp.maximum(dot(x, W1_ref[...]) + b1_ref[...], 0.0)
    q = dot(p, W2_ref[...]) + b2_ref[...]

    r_v = jax.nn.sigmoid(q[:, :128]) * q[:, 128:]

    seg = seg_ref[0, 0, :]
    onehot = (seg[:, None] == jax.lax.broadcasted_iota(
        jnp.int32, (BLOCK, N_GRAPH), 1)).astype(jnp.bfloat16)
    partial = jax.lax.dot_general(onehot, r_v.astype(jnp.bfloat16),
                                  (((0,), (0,)), ((), ())),
                                  preferred_element_type=jnp.float32)

    @pl.when(pl.program_id(0) == 0)
    def _init():
        out_ref[...] = jnp.zeros_like(out_ref)

    out_ref[...] += partial


def kernel(h_T, h_0, graph_index, Wi1, bi1, Wi2, bi2, Wj1, bj1, Wj2, bj2):
    n, d = h_T.shape
    grid = n // BLOCK
    seg3 = graph_index.reshape(grid, 1, BLOCK)

    z = jnp.zeros((d, d), jnp.float32)
    W1 = jnp.concatenate([
        jnp.concatenate([Wi1[:d], Wj1], axis=1),
        jnp.concatenate([Wi1[d:], z], axis=1)], axis=0)
    b1 = jnp.concatenate([bi1, bj1]).reshape(1, -1)
    W2 = jnp.concatenate([
        jnp.concatenate([Wi2, z], axis=1),
        jnp.concatenate([z, Wj2], axis=1)], axis=0)
    b2 = jnp.concatenate([bi2, bj2]).reshape(1, -1)

    row_spec = pl.BlockSpec((BLOCK, d), lambda i: (i, 0))
    full = lambda a: pl.BlockSpec(a.shape, lambda i: (0,) * a.ndim)

    return pl.pallas_call(
        _body,
        grid=(grid,),
        in_specs=[
            pl.BlockSpec((1, 1, BLOCK), lambda i: (i, 0, 0)),
            row_spec, row_spec,
            full(W1), full(b1), full(W2), full(b2),
        ],
        out_specs=pl.BlockSpec((N_GRAPH, d), lambda i: (0, 0)),
        out_shape=jax.ShapeDtypeStruct((N_GRAPH, d), jnp.float32),
        compiler_params=pltpu.CompilerParams(
            dimension_semantics=("arbitrary",)),
    )(seg3, h_T, h_0, W1, b1, W2, b2)


# speedup vs baseline: 1.0329x; 1.0329x over previous
import jax
import jax.numpy as jnp
from jax.experimental import pallas as pl
from jax.experimental.pallas import tpu as pltpu

N_GRAPH = 256
BLOCK = 5000


def _body(seg_ref, hT_ref, h0_ref, W1_ref, b1_ref, W2_ref, b2_ref, out_ref):
    x = jnp.concatenate([hT_ref[...], h0_ref[...]], axis=1)

    def dot(a, b):
        return jax.lax.dot_general(a, b, (((1,), (0,)), ((), ())),
                                   preferred_element_type=jnp.float32)

    p = jnp.maximum(dot(x, W1_ref[...]) + b1_ref[...], 0.0)
    q = dot(p, W2_ref[...]) + b2_ref[...]

    r_v = jax.nn.sigmoid(q[:, :128]) * q[:, 128:]

    seg = seg_ref[0, 0, :]
    onehot = (seg[:, None] == jax.lax.broadcasted_iota(
        jnp.int32, (BLOCK, N_GRAPH), 1)).astype(jnp.bfloat16)
    partial = jax.lax.dot_general(onehot, r_v.astype(jnp.bfloat16),
                                  (((0,), (0,)), ((), ())),
                                  preferred_element_type=jnp.float32)

    @pl.when(pl.program_id(0) == 0)
    def _init():
        out_ref[...] = jnp.zeros_like(out_ref)

    out_ref[...] += partial


def kernel(h_T, h_0, graph_index, Wi1, bi1, Wi2, bi2, Wj1, bj1, Wj2, bj2):
    n, d = h_T.shape
    grid = n // BLOCK
    seg3 = graph_index.reshape(grid, 1, BLOCK)

    z = jnp.zeros((d, d), jnp.float32)
    W1 = jnp.concatenate([
        jnp.concatenate([Wi1[:d], Wj1], axis=1),
        jnp.concatenate([Wi1[d:], z], axis=1)], axis=0)
    b1 = jnp.concatenate([bi1, bj1]).reshape(1, -1)
    W2 = jnp.concatenate([
        jnp.concatenate([Wi2, z], axis=1),
        jnp.concatenate([z, Wj2], axis=1)], axis=0)
    b2 = jnp.concatenate([bi2, bj2]).reshape(1, -1)

    row_spec = pl.BlockSpec((BLOCK, d), lambda i: (i, 0))
    full = lambda a: pl.BlockSpec(a.shape, lambda i: (0,) * a.ndim)

    return pl.pallas_call(
        _body,
        grid=(grid,),
        in_specs=[
            pl.BlockSpec((1, 1, BLOCK), lambda i: (i, 0, 0)),
            row_spec, row_spec,
            full(W1), full(b1), full(W2), full(b2),
        ],
        out_specs=pl.BlockSpec((N_GRAPH, d), lambda i: (0, 0)),
        out_shape=jax.ShapeDtypeStruct((N_GRAPH, d), jnp.float32),
        compiler_params=pltpu.CompilerParams(
            dimension_semantics=("arbitrary",)),
    )(seg3, h_T, h_0, W1, b1, W2, b2)
